# Initial kernel scaffold; baseline (speedup 1.0000x reference)
import jax
import jax.numpy as jnp
from jax.experimental import pallas as pl


def kernel(x, attr_emb, option_embs, prior):
    raise NotImplementedError("write your pallas kernel here")


# speedup vs baseline: 12.5229x; 12.5229x over previous
import functools

import jax
import jax.numpy as jnp
from jax import lax
from jax.experimental import pallas as pl
from jax.experimental.pallas import tpu as pltpu
from jax.experimental.pallas import tpu_sc as plsc

N_ATTRS = 26
VOCAB = 1000
D_MODEL = 128
BATCH = 16384
ROWS = BATCH * N_ATTRS

NC = 2
NS = 16
NW = NC * NS
RPW = ROWS // NW
CHUNK = 128
NCH = RPW // CHUNK


def _fuse_body(prior_ref, opt_ref, attr_ref, out_ref):
    i = pl.program_id(0)
    out_ref[...] = (opt_ref[...] + attr_ref[...]) * prior_ref[i, 0]


def _fused_table(attr_emb, option_embs, prior):
    return pl.pallas_call(
        _fuse_body,
        grid=(N_ATTRS,),
        in_specs=[
            pl.BlockSpec(memory_space=pltpu.SMEM),
            pl.BlockSpec((1, VOCAB, D_MODEL), lambda i: (i, 0, 0)),
            pl.BlockSpec((1, 1, D_MODEL), lambda i: (i, 0, 0)),
        ],
        out_specs=pl.BlockSpec((1, VOCAB, D_MODEL), lambda i: (i, 0, 0)),
        out_shape=jax.ShapeDtypeStruct((N_ATTRS, VOCAB, D_MODEL), jnp.float32),
    )(prior, option_embs, attr_emb.reshape(N_ATTRS, 1, D_MODEL))


def _idx_body(x_ref, out_ref):
    offs = lax.broadcasted_iota(jnp.int32, (BATCH, N_ATTRS), 1) * VOCAB
    out_ref[...] = x_ref[...] + offs


def _flat_idx(x):
    return pl.pallas_call(
        _idx_body,
        out_shape=jax.ShapeDtypeStruct((BATCH, N_ATTRS), jnp.int32),
    )(x)


_mesh = plsc.VectorSubcoreMesh(core_axis_name="c", subcore_axis_name="s")


@functools.partial(
    pl.kernel,
    mesh=_mesh,
    out_type=jax.ShapeDtypeStruct((ROWS, D_MODEL), jnp.float32),
    scratch_types=[
        pltpu.VMEM((RPW,), jnp.int32),
        pltpu.VMEM((CHUNK, D_MODEL), jnp.float32),
        pltpu.VMEM((CHUNK, D_MODEL), jnp.float32),
        pltpu.SemaphoreType.DMA,
        pltpu.SemaphoreType.DMA,
        pltpu.SemaphoreType.DMA,
        pltpu.SemaphoreType.DMA,
    ],
)
def _gather_kernel(table_hbm, idx_hbm, out_hbm, idx_v, buf0, buf1,
                   g0, g1, s0, s1):
    wid = lax.axis_index("s") * NC + lax.axis_index("c")
    base = wid * RPW
    pltpu.sync_copy(idx_hbm.at[pl.ds(base, RPW)], idx_v)

    def start_gather(c, buf, sem):
        pltpu.async_copy(table_hbm.at[idx_v.at[pl.ds(c * CHUNK, CHUNK)]],
                         buf, sem)

    def wait_gather(buf, sem):
        pltpu.make_async_copy(table_hbm.at[pl.ds(0, CHUNK)], buf, sem).wait()

    def start_put(c, buf, sem):
        pltpu.async_copy(buf, out_hbm.at[pl.ds(base + c * CHUNK, CHUNK)], sem)

    def wait_put(c, buf, sem):
        pltpu.make_async_copy(
            buf, out_hbm.at[pl.ds(base + c * CHUNK, CHUNK)], sem).wait()

    start_gather(0, buf0, g0)
    start_gather(1, buf1, g1)

    def body(p, carry):
        c = 2 * p
        wait_gather(buf0, g0)
        start_put(c, buf0, s0)
        wait_put(c, buf0, s0)
        start_gather(c + 2, buf0, g0)
        wait_gather(buf1, g1)
        start_put(c + 1, buf1, s1)
        wait_put(c + 1, buf1, s1)
        start_gather(c + 3, buf1, g1)
        return carry

    lax.fori_loop(0, NCH // 2 - 1, body, 0)

    c_last = NCH - 2
    wait_gather(buf0, g0)
    start_put(c_last, buf0, s0)
    wait_gather(buf1, g1)
    start_put(c_last + 1, buf1, s1)
    wait_put(c_last, buf0, s0)
    wait_put(c_last + 1, buf1, s1)


def kernel(x, attr_emb, option_embs, prior):
    x = x.astype(jnp.int32)
    table = _fused_table(attr_emb, option_embs, prior)
    idx = _flat_idx(x).reshape(ROWS)
    out = _gather_kernel(table.reshape(N_ATTRS * VOCAB, D_MODEL), idx)
    return out.reshape(BATCH, N_ATTRS, D_MODEL)


# speedup vs baseline: 20.3881x; 1.6281x over previous
import functools

import jax
import jax.numpy as jnp
from jax import lax
from jax.experimental import pallas as pl
from jax.experimental.pallas import tpu as pltpu
from jax.experimental.pallas import tpu_sc as plsc

N_ATTRS = 26
VOCAB = 1000
D_MODEL = 128
BATCH = 16384
ROWS = BATCH * N_ATTRS

NC = 2
NS = 16
NW = NC * NS
RPW = ROWS // NW
CHUNK = 128
NCH = RPW // CHUNK


def _fuse_body(prior_ref, opt_ref, attr_ref, out_ref):
    i = pl.program_id(0)
    out_ref[...] = (opt_ref[...] + attr_ref[...]) * prior_ref[i, 0]


def _fused_table(attr_emb, option_embs, prior):
    return pl.pallas_call(
        _fuse_body,
        grid=(N_ATTRS,),
        in_specs=[
            pl.BlockSpec(memory_space=pltpu.SMEM),
            pl.BlockSpec((1, VOCAB, D_MODEL), lambda i: (i, 0, 0)),
            pl.BlockSpec((1, 1, D_MODEL), lambda i: (i, 0, 0)),
        ],
        out_specs=pl.BlockSpec((1, VOCAB, D_MODEL), lambda i: (i, 0, 0)),
        out_shape=jax.ShapeDtypeStruct((N_ATTRS, VOCAB, D_MODEL), jnp.float32),
    )(prior, option_embs, attr_emb.reshape(N_ATTRS, 1, D_MODEL))


def _idx_body(x_ref, out_ref):
    offs = lax.broadcasted_iota(jnp.int32, (BATCH, N_ATTRS), 1) * VOCAB
    out_ref[...] = x_ref[...] + offs


def _flat_idx(x):
    return pl.pallas_call(
        _idx_body,
        out_shape=jax.ShapeDtypeStruct((BATCH, N_ATTRS), jnp.int32),
    )(x)


BPW = BATCH // NW
CB = 16
CROWS = CB * N_ATTRS
GROWS = 104
GPC = CROWS // GROWS
NCHUNK = BPW // CB

_mesh = plsc.VectorSubcoreMesh(core_axis_name="c", subcore_axis_name="s")


@functools.partial(
    pl.kernel,
    mesh=_mesh,
    out_type=jax.ShapeDtypeStruct((BATCH, N_ATTRS, D_MODEL), jnp.float32),
    scratch_types=[
        pltpu.VMEM((RPW,), jnp.int32),
        pltpu.VMEM((CROWS, D_MODEL), jnp.float32),
        pltpu.VMEM((CROWS, D_MODEL), jnp.float32),
        pltpu.SemaphoreType.DMA,
        pltpu.SemaphoreType.DMA,
        pltpu.SemaphoreType.DMA,
        pltpu.SemaphoreType.DMA,
    ],
)
def _gather_kernel(table_hbm, idx_hbm, out_hbm, idx_v, buf0, buf1,
                   g0, g1, s0, s1):
    wid = lax.axis_index("s") * NC + lax.axis_index("c")
    rbase = wid * RPW
    bbase = wid * BPW
    pltpu.sync_copy(idx_hbm.at[pl.ds(rbase, RPW)], idx_v)

    def start_gathers(c, buf, sem):
        for g in range(GPC):
            pltpu.async_copy(
                table_hbm.at[idx_v.at[pl.ds(c * CROWS + g * GROWS, GROWS)]],
                buf.at[pl.ds(g * GROWS, GROWS)], sem)

    def wait_gathers(buf, sem):
        pltpu.make_async_copy(table_hbm.at[pl.ds(0, CROWS)], buf, sem).wait()

    def start_put(c, buf, sem):
        pltpu.async_copy(buf.reshape(CB, N_ATTRS, D_MODEL),
                         out_hbm.at[pl.ds(bbase + c * CB, CB)], sem)

    def wait_put(c, buf, sem):
        pltpu.make_async_copy(buf.reshape(CB, N_ATTRS, D_MODEL),
                              out_hbm.at[pl.ds(bbase + c * CB, CB)],
                              sem).wait()

    start_gathers(0, buf0, g0)
    start_gathers(1, buf1, g1)

    def body(p, carry):
        c = 2 * p
        wait_gathers(buf0, g0)
        start_put(c, buf0, s0)
        wait_put(c, buf0, s0)
        start_gathers(c + 2, buf0, g0)
        wait_gathers(buf1, g1)
        start_put(c + 1, buf1, s1)
        wait_put(c + 1, buf1, s1)
        start_gathers(c + 3, buf1, g1)
        return carry

    lax.fori_loop(0, NCHUNK // 2 - 1, body, 0)

    c_last = NCHUNK - 2
    wait_gathers(buf0, g0)
    start_put(c_last, buf0, s0)
    wait_gathers(buf1, g1)
    start_put(c_last + 1, buf1, s1)
    wait_put(c_last, buf0, s0)
    wait_put(c_last + 1, buf1, s1)


def kernel(x, attr_emb, option_embs, prior):
    x = x.astype(jnp.int32)
    table = _fused_table(attr_emb, option_embs, prior)
    idx = _flat_idx(x).reshape(ROWS)
    return _gather_kernel(table.reshape(N_ATTRS * VOCAB, D_MODEL), idx)
